# baseline (speedup 1.0000x reference)
import functools

import jax
import jax.numpy as jnp
from jax.experimental import pallas as pl
from jax.experimental.pallas import tpu as pltpu

B, T, D = 2, 2048, 1024
E, H, K = 8, 512, 2
BT = B * T
CHUNK = 1024
NC = BT // CHUNK


def _moe_body(x_ref, selT_ref, k_ref, v_ref, rs_ref, o_ref, w_ref):
    j = pl.program_id(0)

    @pl.when(j == 0)
    def _router():
        x = x_ref[...]
        logits = jnp.dot(x, selT_ref[...], preferred_element_type=jnp.float32)
        p = jax.nn.sigmoid(logits)
        eidx = jax.lax.broadcasted_iota(jnp.int32, (BT, E), 1)
        m1 = jnp.max(p, axis=1, keepdims=True)
        a1 = jnp.min(jnp.where(p == m1, eidx, E), axis=1, keepdims=True)
        p2 = jnp.where(eidx == a1, -1.0, p)
        m2 = jnp.max(p2, axis=1, keepdims=True)
        a2 = jnp.min(jnp.where(p2 == m2, eidx, E), axis=1, keepdims=True)
        selm = (eidx == a1) | (eidx == a2)
        denom = jnp.maximum(m1 + m2, 1e-9)
        w_ref[...] = jnp.where(selm, p / denom * rs_ref[0], 0.0)
        o_ref[...] = jnp.zeros((BT, D), jnp.float32)

    kb = k_ref[0].astype(jnp.bfloat16)
    vb = v_ref[0].astype(jnp.bfloat16)
    eidx = jax.lax.broadcasted_iota(jnp.int32, (CHUNK, E), 1)
    for c in range(NC):
        sl = pl.ds(c * CHUNK, CHUNK)
        xc = x_ref[sl, :].astype(jnp.bfloat16)
        h = jnp.dot(xc, kb, preferred_element_type=jnp.float32)
        wc = w_ref[sl, :]
        wj = jnp.sum(jnp.where(eidx == j, wc, 0.0), axis=1, keepdims=True)
        hs = (jnp.maximum(h, 0.0) * wj).astype(jnp.bfloat16)
        o_ref[sl, :] += jnp.dot(hs, vb, preferred_element_type=jnp.float32)


@functools.partial(jax.jit, static_argnames=("interpret",))
def _moe(x2d, selT, keys, values, route_scale, interpret=False):
    out = pl.pallas_call(
        _moe_body,
        grid=(E,),
        in_specs=[
            pl.BlockSpec((BT, D), lambda j: (0, 0)),
            pl.BlockSpec((D, E), lambda j: (0, 0)),
            pl.BlockSpec((1, D, H), lambda j: (j, 0, 0)),
            pl.BlockSpec((1, H, D), lambda j: (j, 0, 0)),
            pl.BlockSpec(memory_space=pltpu.SMEM),
        ],
        out_specs=pl.BlockSpec((BT, D), lambda j: (0, 0)),
        out_shape=jax.ShapeDtypeStruct((BT, D), jnp.float32),
        scratch_shapes=[
            pltpu.VMEM((BT, E), jnp.float32),
        ],
        interpret=interpret,
    )(x2d, selT, keys, values, route_scale)
    return out


def kernel(input, expert_sel, keys, values, route_scale, interpret=False):
    x2d = input.reshape(BT, D)
    selT = expert_sel.T
    out = _moe(x2d, selT, keys, values, route_scale, interpret=interpret)
    return out.reshape(B, T, D)


# speedup vs baseline: 1.0022x; 1.0022x over previous
import functools

import jax
import jax.numpy as jnp
from jax.experimental import pallas as pl
from jax.experimental.pallas import tpu as pltpu

B, T, D = 2, 2048, 1024
E, H, K = 8, 512, 2
BT = B * T
CHUNK = 1024
NC = BT // CHUNK


def _moe_body(x_ref, selT_ref, k_ref, v_ref, rs_ref, o_ref, w_ref):
    j = pl.program_id(0)

    @pl.when(j == 0)
    def _router():
        x = x_ref[...]
        logits = jnp.dot(x, selT_ref[...], preferred_element_type=jnp.float32)
        p = jax.nn.sigmoid(logits)
        eidx = jax.lax.broadcasted_iota(jnp.int32, (BT, E), 1)
        m1 = jnp.max(p, axis=1, keepdims=True)
        a1 = jnp.min(jnp.where(p == m1, eidx, E), axis=1, keepdims=True)
        p2 = jnp.where(eidx == a1, -1.0, p)
        m2 = jnp.max(p2, axis=1, keepdims=True)
        a2 = jnp.min(jnp.where(p2 == m2, eidx, E), axis=1, keepdims=True)
        selm = (eidx == a1) | (eidx == a2)
        denom = jnp.maximum(m1 + m2, 1e-9)
        w_ref[...] = jnp.where(selm, p / denom * rs_ref[0], 0.0)
        o_ref[...] = jnp.zeros((BT, D), jnp.float32)

    kb = k_ref[0]
    vb = v_ref[0]
    eidx = jax.lax.broadcasted_iota(jnp.int32, (CHUNK, E), 1)
    for c in range(NC):
        sl = pl.ds(c * CHUNK, CHUNK)
        xc = x_ref[sl, :]
        h = jnp.dot(xc, kb, preferred_element_type=jnp.float32)
        wc = w_ref[sl, :]
        wj = jnp.sum(jnp.where(eidx == j, wc, 0.0), axis=1, keepdims=True)
        hs = jnp.maximum(h, 0.0) * wj
        o_ref[sl, :] += jnp.dot(hs, vb, preferred_element_type=jnp.float32)


@jax.jit
def _moe(x2d, selT, keys, values, route_scale):
    out = pl.pallas_call(
        _moe_body,
        grid=(E,),
        in_specs=[
            pl.BlockSpec((BT, D), lambda j: (0, 0)),
            pl.BlockSpec((D, E), lambda j: (0, 0)),
            pl.BlockSpec((1, D, H), lambda j: (j, 0, 0)),
            pl.BlockSpec((1, H, D), lambda j: (j, 0, 0)),
            pl.BlockSpec(memory_space=pltpu.SMEM),
        ],
        out_specs=pl.BlockSpec((BT, D), lambda j: (0, 0)),
        out_shape=jax.ShapeDtypeStruct((BT, D), jnp.float32),
        scratch_shapes=[
            pltpu.VMEM((BT, E), jnp.float32),
        ],
    )(x2d, selT, keys, values, route_scale)
    return out


def kernel(input, expert_sel, keys, values, route_scale):
    x2d = input.reshape(BT, D)
    selT = expert_sel.T
    out = _moe(x2d, selT, keys, values, route_scale)
    return out.reshape(B, T, D)
